# Initial kernel scaffold; baseline (speedup 1.0000x reference)
import jax
import jax.numpy as jnp
from jax.experimental import pallas as pl


def kernel(logit, target, log_cls_num, k_per_class):
    raise NotImplementedError("write your pallas kernel here")


# speedup vs baseline: 11.8801x; 11.8801x over previous
import jax
import jax.numpy as jnp
from jax.experimental import pallas as pl

_B = 4096
_C = 1000
_BR = 256
_NB = _B // _BR


def _tc_body(x_ref, lcn_ref, tgt_ref, kpc_ref, out_ref):
    x = x_ref[...]
    lcn = lcn_ref[...]
    tgt = tgt_ref[...]
    kpc = kpc_ref[...]

    la = x + lcn
    m = jnp.max(la, axis=1, keepdims=True)
    e = jnp.exp(la - m)
    z = jnp.sum(e, axis=1, keepdims=True)

    cols = jax.lax.broadcasted_iota(jnp.int32, (_BR, _C), 1)
    oh = cols == tgt
    la_t = jnp.sum(jnp.where(oh, la, 0.0), axis=1, keepdims=True)
    x_t = jnp.sum(jnp.where(oh, x, 0.0), axis=1, keepdims=True)
    k_t = jnp.sum(jnp.where(oh, jnp.broadcast_to(kpc, (_BR, _C)), 0),
                  axis=1, keepdims=True)
    k_t = jnp.minimum(k_t, _C)

    xb = jax.lax.bitcast_convert_type(x, jnp.int32)
    key = xb ^ (jax.lax.shift_right_arithmetic(xb, 31) & jnp.int32(0x7FFFFFFF))
    xtb = jax.lax.bitcast_convert_type(x_t, jnp.int32)
    key_t = xtb ^ (jax.lax.shift_right_arithmetic(xtb, 31)
                   & jnp.int32(0x7FFFFFFF))

    def bit_step(i, prefix):
        shift = jnp.int32(31) - i
        trial = prefix + (jnp.int32(1) << shift)
        cnt = jnp.sum(jnp.where(key >= trial, jnp.int32(1), jnp.int32(0)),
                      axis=1, keepdims=True)
        return jnp.where(cnt >= k_t, trial, prefix)

    prefix0 = jnp.full((_BR, 1), jnp.int32(-(2 ** 31)))
    thr = jax.lax.fori_loop(0, 32, bit_step, prefix0)

    s_num = jnp.sum(jnp.where(key >= thr, e, 0.0), axis=1, keepdims=True)
    in_t = key_t >= thr

    log_z = m + jnp.log(z)
    lf = log_z - la_t
    p_t = jnp.exp(la_t - log_z)
    num = jnp.where(in_t, p_t + jnp.float32(1e-6), jnp.float32(1e-6))
    lt = jnp.log(s_num / z + jnp.float32(_C * 1e-6)) - jnp.log(num)
    tot = jnp.sum(0.5 * (lf + lt))
    out_ref[...] = jnp.full((1, 8, 128), tot / jnp.float32(1024.0))


def kernel(logit, target, log_cls_num, k_per_class):
    lcn2 = log_cls_num.reshape(1, _C)
    tgt2 = target.reshape(_B, 1)
    kpc2 = k_per_class.reshape(1, _C)
    out = pl.pallas_call(
        _tc_body,
        grid=(_NB,),
        in_specs=[
            pl.BlockSpec((_BR, _C), lambda i: (i, 0)),
            pl.BlockSpec((1, _C), lambda i: (0, 0)),
            pl.BlockSpec((_BR, 1), lambda i: (i, 0)),
            pl.BlockSpec((1, _C), lambda i: (0, 0)),
        ],
        out_specs=pl.BlockSpec((1, 8, 128), lambda i: (i, 0, 0)),
        out_shape=jax.ShapeDtypeStruct((_NB, 8, 128), jnp.float32),
    )(logit, lcn2, tgt2, kpc2)
    return jnp.sum(out) / jnp.float32(_B)
